# baseline (speedup 1.0000x reference)
import functools
import math

import jax
import jax.numpy as jnp
from jax import lax
from jax.experimental import pallas as pl
from jax.experimental.pallas import tpu as pltpu
from jax.experimental.pallas import tpu_sc as plsc

VOCAB = 1000000
D_MODEL = 64
SCALE = math.sqrt(D_MODEL)

NC = 2
NS = 16
NW = NC * NS

IDX_ROW = 128
ROWS_PER_CHUNK = 8
CHUNK = IDX_ROW * ROWS_PER_CHUNK


def _emb_body(idx_hbm, table_hbm, out_hbm, idx_v, rows_v, sem, *, chunks_per_w):
    wid = lax.axis_index("s") * NC + lax.axis_index("c")

    def chunk_body(ci, _):
        chunk = wid * chunks_per_w + ci
        pltpu.sync_copy(idx_hbm.at[pl.ds(chunk * ROWS_PER_CHUNK, ROWS_PER_CHUNK)],
                        idx_v)
        copies = [
            pltpu.async_copy(table_hbm.at[idx_v.at[j]],
                             rows_v.at[pl.ds(j * IDX_ROW, IDX_ROW)], sem)
            for j in range(ROWS_PER_CHUNK)
        ]
        for c in copies:
            c.wait()

        def scale_row(r, _):
            for j in range(D_MODEL // 16):
                sl = pl.ds(j * 16, 16)
                rows_v[r, sl] = rows_v[r, sl] * SCALE
            return ()

        lax.fori_loop(0, CHUNK, scale_row, (), unroll=4)

        pltpu.sync_copy(rows_v, out_hbm.at[pl.ds(chunk * CHUNK, CHUNK)])
        return ()

    lax.fori_loop(0, chunks_per_w, chunk_body, ())


def kernel(x, table):
    B = x.shape[0] * x.shape[1]
    assert B % (NW * CHUNK) == 0
    chunks_per_w = B // (NW * CHUNK)

    idx = x.reshape(B // IDX_ROW, IDX_ROW).astype(jnp.int32)
    mesh = plsc.VectorSubcoreMesh(core_axis_name="c", subcore_axis_name="s")

    emb = pl.kernel(
        functools.partial(_emb_body, chunks_per_w=chunks_per_w),
        out_type=jax.ShapeDtypeStruct((B, D_MODEL), jnp.float32),
        mesh=mesh,
        scratch_types=[
            pltpu.VMEM((ROWS_PER_CHUNK, IDX_ROW), jnp.int32),
            pltpu.VMEM((CHUNK, D_MODEL), jnp.float32),
            pltpu.SemaphoreType.DMA,
        ],
        compiler_params=pltpu.CompilerParams(use_tc_tiling_on_sc=False),
    )(idx, table)
    return emb.reshape(x.shape[0], x.shape[1], D_MODEL)


# speedup vs baseline: 1.0514x; 1.0514x over previous
import functools
import math

import jax
import jax.numpy as jnp
from jax import lax
from jax.experimental import pallas as pl
from jax.experimental.pallas import tpu as pltpu
from jax.experimental.pallas import tpu_sc as plsc

VOCAB = 1000000
D_MODEL = 64
SCALE = math.sqrt(D_MODEL)

NC = 2
NS = 16
NW = NC * NS

IDX_ROW = 128
GATHERS = 5
CHUNK = IDX_ROW * GATHERS


def _emb_body(idx_hbm, table_hbm, out_hbm, idx_v, rows_v, sems, *,
              chunks_per_w):
    wid = lax.axis_index("s") * NC + lax.axis_index("c")
    base_chunk = wid * chunks_per_w

    def issue(ci, b):
        chunk = base_chunk + ci
        pltpu.sync_copy(idx_hbm.at[pl.ds(chunk * GATHERS, GATHERS)], idx_v[b])
        for j in range(GATHERS):
            pltpu.async_copy(table_hbm.at[idx_v[b].at[j]],
                             rows_v[b].at[pl.ds(j * IDX_ROW, IDX_ROW)],
                             sems[b])

    def drain(b):
        pltpu.make_async_copy(table_hbm.at[pl.ds(0, CHUNK)], rows_v[b],
                              sems[b]).wait()

    def scale_and_store(ci, b):
        rows = rows_v[b]

        @plsc.parallel_loop(0, CHUNK, step=1, unroll=8)
        def _(r):
            for j in range(D_MODEL // 16):
                sl = pl.ds(j * 16, 16)
                rows[r, sl] = rows[r, sl] * SCALE

        pltpu.sync_copy(rows, out_hbm.at[pl.ds((base_chunk + ci) * CHUNK,
                                               CHUNK)])

    issue(0, 0)

    def pair_body(cp, _):
        for b in range(2):
            ci = cp * 2 + b

            @pl.when(ci + 1 < chunks_per_w)
            def _():
                issue(ci + 1, 1 - b)

            drain(b)
            scale_and_store(ci, b)
        return ()

    lax.fori_loop(0, chunks_per_w // 2, pair_body, ())


def kernel(x, table):
    B = x.shape[0] * x.shape[1]
    assert B % (NW * CHUNK) == 0 and (B // (NW * CHUNK)) % 2 == 0
    chunks_per_w = B // (NW * CHUNK)

    idx = x.reshape(B // IDX_ROW, IDX_ROW).astype(jnp.int32)
    mesh = plsc.VectorSubcoreMesh(core_axis_name="c", subcore_axis_name="s")

    emb = pl.kernel(
        functools.partial(_emb_body, chunks_per_w=chunks_per_w),
        out_type=jax.ShapeDtypeStruct((B, D_MODEL), jnp.float32),
        mesh=mesh,
        scratch_types=[
            [pltpu.VMEM((GATHERS, IDX_ROW), jnp.int32) for _ in range(2)],
            [pltpu.VMEM((CHUNK, D_MODEL), jnp.float32) for _ in range(2)],
            [pltpu.SemaphoreType.DMA for _ in range(2)],
        ],
        compiler_params=pltpu.CompilerParams(use_tc_tiling_on_sc=False),
    )(idx, table)
    return emb.reshape(x.shape[0], x.shape[1], D_MODEL)
